# baseline (speedup 1.0000x reference)
import jax
import jax.numpy as jnp
from jax import lax
from jax.experimental import pallas as pl

_N = 4096
_K = 2048


def _pbit(b):
    return b + 3 if b < 9 else b - 9


def _sort_block(x_ref, o_ref):
    T = x_ref[...]
    C = T.shape[1]
    sidx = lax.broadcasted_iota(jnp.int32, (_N, 1), 0)

    def slice_round(Tc, H, D):
        G = H // (2 * D)
        Tr = Tc.reshape(G, 2, D, C)
        na = jnp.maximum(Tr[:, 0], Tr[:, 1])
        nb = jnp.minimum(Tr[:, 0], Tr[:, 1])
        return jnp.concatenate([na[:, None], nb[:, None]], axis=1).reshape(H, C)

    def roll_round(Tc, idx, D):
        up = jnp.concatenate([Tc[D:], Tc[:D]], axis=0)
        dn = jnp.concatenate([Tc[-D:], Tc[:-D]], axis=0)
        is_a = (idx & D) == 0
        partner = jnp.where(is_a, up, dn)
        mn = jnp.minimum(Tc, partner)
        mx = jnp.maximum(Tc, partner)
        return jnp.where(is_a, mx, mn)

    for st in range(1, 9):
        for j in range(st - 1, -1, -1):
            Dj = 1 << (j + 3)
            A = _N >> (st + 4)
            Bm = 1 << (st - j - 1)
            Tr = T.reshape(A, 2, Bm, 2, Dj, C)
            d = Tr[:, 0]
            e = Tr[:, 1]
            nda = jnp.maximum(d[:, :, 0], d[:, :, 1])
            ndb = jnp.minimum(d[:, :, 0], d[:, :, 1])
            nea = jnp.minimum(e[:, :, 0], e[:, :, 1])
            neb = jnp.maximum(e[:, :, 0], e[:, :, 1])
            nd = jnp.concatenate([nda[:, :, None], ndb[:, :, None]], axis=2)
            ne = jnp.concatenate([nea[:, :, None], neb[:, :, None]], axis=2)
            T = jnp.concatenate([nd[:, None], ne[:, None]], axis=1)
            T = T.reshape(_N, C)

    cur = (sidx >> _pbit(9)) & 1
    T = jnp.where(cur == 1, -T, T)
    for st in range(9, 12):
        for j in range(st - 1, -1, -1):
            D = 1 << _pbit(j)
            if D >= 8:
                T = slice_round(T, _N, D)
            else:
                T = roll_round(T, sidx, D)
        nxt = ((sidx >> _pbit(st + 1)) & 1) if st < 11 else jnp.zeros_like(sidx)
        flip = cur ^ nxt
        T = jnp.where(flip == 1, -T, T)
        cur = nxt

    Tr = T.reshape(512, 2, 4, C)
    T2 = jnp.maximum(Tr[:, 0], Tr[:, 1]).reshape(_K, C)
    s2idx = lax.broadcasted_iota(jnp.int32, (_K, 1), 0)
    for j in range(10, -1, -1):
        if j >= 9:
            T2 = roll_round(T2, s2idx, 1 << (j - 9))
        elif j >= 1:
            T2 = slice_round(T2, _K, 1 << (j + 2))
        else:
            T2 = roll_round(T2, s2idx, 4)

    o_ref[...] = T2.reshape(512, 4, C)


def kernel(x):
    B, CH, N = x.shape
    R = B * CH
    xr = x.reshape(R, 8, 512).transpose(2, 1, 0).reshape(_N, R)
    C = 256
    out = pl.pallas_call(
        _sort_block,
        grid=(R // C,),
        in_specs=[pl.BlockSpec((_N, C), lambda g: (0, g))],
        out_specs=pl.BlockSpec((_N // 8, 4, C), lambda g: (0, 0, g)),
        out_shape=jax.ShapeDtypeStruct((_N // 8, 4, R), jnp.float32),
    )(xr)
    y = out.transpose(2, 1, 0).reshape(R, _K)
    return y.reshape(B, CH, _K)


# speedup vs baseline: 1.1798x; 1.1798x over previous
import jax
import jax.numpy as jnp
from jax import lax
from jax.experimental import pallas as pl

_N = 4096
_K = 2048


def _pbit(b):
    return b + 3 if b < 9 else b - 9


def _sort_block(x_ref, o_ref):
    T = x_ref[...]
    C = T.shape[1]
    sidx = lax.broadcasted_iota(jnp.int32, (_N, 1), 0)

    def slice_round(Tc, H, D):
        G = H // (2 * D)
        Tr = Tc.reshape(G, 2, D, C)
        na = jnp.maximum(Tr[:, 0], Tr[:, 1])
        nb = jnp.minimum(Tr[:, 0], Tr[:, 1])
        return jnp.concatenate([na[:, None], nb[:, None]], axis=1).reshape(H, C)

    def roll_round(Tc, idx, D):
        up = jnp.concatenate([Tc[D:], Tc[:D]], axis=0)
        dn = jnp.concatenate([Tc[-D:], Tc[:-D]], axis=0)
        is_a = (idx & D) == 0
        partner = jnp.where(is_a, up, dn)
        mn = jnp.minimum(Tc, partner)
        mx = jnp.maximum(Tc, partner)
        return jnp.where(is_a, mx, mn)

    for st in range(1, 9):
        for j in range(st - 1, -1, -1):
            Dj = 1 << (j + 3)
            A = _N >> (st + 4)
            Bm = 1 << (st - j - 1)
            Tr = T.reshape(A, 2, Bm, 2, Dj, C)
            nda = jnp.maximum(Tr[:, 0, :, 0], Tr[:, 0, :, 1])
            ndb = jnp.minimum(Tr[:, 0, :, 0], Tr[:, 0, :, 1])
            nea = jnp.minimum(Tr[:, 1, :, 0], Tr[:, 1, :, 1])
            neb = jnp.maximum(Tr[:, 1, :, 0], Tr[:, 1, :, 1])
            nd = jnp.concatenate([nda[:, :, None], ndb[:, :, None]], axis=2)
            ne = jnp.concatenate([nea[:, :, None], neb[:, :, None]], axis=2)
            T = jnp.concatenate([nd[:, None], ne[:, None]], axis=1)
            T = T.reshape(_N, C)

    cur = (sidx >> _pbit(9)) & 1
    T = jnp.where(cur == 1, -T, T)
    for st in range(9, 12):
        for j in range(st - 1, -1, -1):
            D = 1 << _pbit(j)
            if D >= 8:
                T = slice_round(T, _N, D)
            else:
                T = roll_round(T, sidx, D)
        nxt = ((sidx >> _pbit(st + 1)) & 1) if st < 11 else jnp.zeros_like(sidx)
        flip = cur ^ nxt
        T = jnp.where(flip == 1, -T, T)
        cur = nxt

    Tr = T.reshape(512, 2, 4, C)
    T2 = jnp.maximum(Tr[:, 0], Tr[:, 1]).reshape(_K, C)
    s2idx = lax.broadcasted_iota(jnp.int32, (_K, 1), 0)
    for j in range(10, -1, -1):
        if j >= 9:
            T2 = roll_round(T2, s2idx, 1 << (j - 9))
        elif j >= 1:
            T2 = slice_round(T2, _K, 1 << (j + 2))
        else:
            T2 = roll_round(T2, s2idx, 4)

    o_ref[...] = T2.reshape(512, 4, C)


def kernel(x):
    B, CH, N = x.shape
    R = B * CH
    xr = x.reshape(R, 8, 512).transpose(2, 1, 0).reshape(_N, R)
    C = 128
    out = pl.pallas_call(
        _sort_block,
        grid=(R // C,),
        in_specs=[pl.BlockSpec((_N, C), lambda g: (0, g))],
        out_specs=pl.BlockSpec((_N // 8, 4, C), lambda g: (0, 0, g)),
        out_shape=jax.ShapeDtypeStruct((_N // 8, 4, R), jnp.float32),
    )(xr)
    y = out.transpose(2, 1, 0).reshape(R, _K)
    return y.reshape(B, CH, _K)
